# baseline (speedup 1.0000x reference)
import functools

import jax
import jax.numpy as jnp
from jax import lax
from jax.experimental import pallas as pl
from jax.experimental.pallas import tpu as pltpu
from jax.experimental.pallas import tpu_sc as plsc

_BATCH = 4
_SEQ = 2048
_DIMS = 1024

_info = plsc.get_sparse_core_info()
_NC, _NS = _info.num_cores, _info.num_subcores
_NW = _NC * _NS
_ROWS_PER_W = _SEQ // _NW
_R_SH = 56


def _make_gather_broadcast():
  mesh = plsc.VectorSubcoreMesh(core_axis_name="c", subcore_axis_name="s")

  @functools.partial(
      pl.kernel,
      mesh=mesh,
      out_type=jax.ShapeDtypeStruct((_BATCH, _SEQ, _DIMS), jnp.float32),
      scratch_types=[
          pltpu.VMEM((_ROWS_PER_W,), jnp.int32),
          pltpu.VMEM((_ROWS_PER_W, _DIMS), jnp.float32),
          pltpu.VMEM_SHARED((_NS, _R_SH, _DIMS), jnp.float32),
          pltpu.SemaphoreType.DMA,
          pltpu.SemaphoreType.DMA,
          pltpu.SemaphoreType.DMA,
          pltpu.SemaphoreType.DMA,
      ],
  )
  def gather_broadcast(table_hbm, idx_hbm, out_hbm, idx_v, rows_v, shared,
                       sem_g, sem_d, sem_w, sem_s):
    cid = lax.axis_index("c")
    sid = lax.axis_index("s")
    wid = sid * _NC + cid
    base = wid * _ROWS_PER_W
    pltpu.sync_copy(idx_hbm.at[pl.ds(base, _ROWS_PER_W)], idx_v)
    pltpu.async_copy(table_hbm.at[idx_v], rows_v, sem_g).wait()
    deposit = pltpu.async_copy(rows_v.at[pl.ds(0, _R_SH)], shared.at[sid],
                               sem_d)
    writes = [
        pltpu.async_copy(rows_v, out_hbm.at[b, pl.ds(base, _ROWS_PER_W)],
                         sem_w)
        for b in range(2)
    ] + [
        pltpu.async_copy(rows_v.at[pl.ds(_R_SH, _ROWS_PER_W - _R_SH)],
                         out_hbm.at[b, pl.ds(base + _R_SH,
                                             _ROWS_PER_W - _R_SH)], sem_w)
        for b in range(2, _BATCH)
    ]
    deposit.wait()
    plsc.subcore_barrier()

    @pl.when(sid == 0)
    def _():
      drains = [
          pltpu.async_copy(
              shared.at[s],
              out_hbm.at[b, pl.ds((s * _NC + cid) * _ROWS_PER_W, _R_SH)],
              sem_s,
          )
          for b in range(2, _BATCH)
          for s in range(_NS)
      ]
      for d in drains:
        d.wait()

    for w in writes:
      w.wait()

  return gather_broadcast


_gather_broadcast = _make_gather_broadcast()


def kernel(x, E_absolute_position, relative_index):
  del x
  return _gather_broadcast(E_absolute_position, relative_index)
